# Initial kernel scaffold; baseline (speedup 1.0000x reference)
import jax
import jax.numpy as jnp
from jax.experimental import pallas as pl


def kernel(pred_tensor, target_tensor):
    raise NotImplementedError("write your pallas kernel here")


# speedup vs baseline: 3.2567x; 3.2567x over previous
import functools

import jax
import jax.numpy as jnp
from jax import lax
from jax.experimental import pallas as pl
from jax.experimental.pallas import tpu as pltpu
from jax.experimental.pallas import tpu_sc as plsc

S = 7.0
NCH = 30
N_ROWS = 4096 * 7 * 7
NW = 32
ROWS_PER_W = N_ROWS // NW
CHUNK_ROWS = 448
N_CHUNKS = ROWS_PER_W // CHUNK_ROWS
CHUNK_WORDS = CHUNK_ROWS * NCH
GROUPS = CHUNK_ROWS // 16


def _sqrt16(x):
    i = lax.bitcast_convert_type(x, jnp.int32)
    y = lax.bitcast_convert_type(jnp.int32(0x5F3759DF) - (i >> 1), jnp.float32)
    for _ in range(3):
        y = y * (1.5 - 0.5 * x * y * y)
    return x * y


def _xyxy(x, y, w, h):
    cx = x / S
    cy = y / S
    return cx - 0.5 * w, cy - 0.5 * h, cx + 0.5 * w, cy + 0.5 * h


def _body(p_hbm, t_hbm, out_hbm, pb0, pb1, tb0, tb1, oidx, obuf,
          sp0, sp1, st0, st1):
    wid = lax.axis_index("s") * 2 + lax.axis_index("c")
    base_word = wid * (ROWS_PER_W * NCH)
    pbufs = (pb0, pb1)
    tbufs = (tb0, tb1)
    psems = (sp0, sp1)
    tsems = (st0, st1)

    def start(c):
        slot = c % 2
        off = base_word + c * CHUNK_WORDS
        cp = pltpu.async_copy(p_hbm.at[pl.ds(off, CHUNK_WORDS)], pbufs[slot],
                              psems[slot])
        ct = pltpu.async_copy(t_hbm.at[pl.ds(off, CHUNK_WORDS)], tbufs[slot],
                              tsems[slot])
        return cp, ct

    handles = {0: start(0)}
    iota = lax.iota(jnp.int32, 16)
    row_base = iota * NCH
    acc_no = jnp.zeros((16,), jnp.float32)
    acc_hv = jnp.zeros((16,), jnp.float32)

    for c in range(N_CHUNKS):
        if c + 1 < N_CHUNKS:
            handles[c + 1] = start(c + 1)
        cp, ct = handles.pop(c)
        cp.wait()
        ct.wait()
        slot = c % 2
        pb = pbufs[slot]
        tb = tbufs[slot]

        def phase_a(g, carry):
            accn, cnt = carry
            idx = row_base + g * (16 * NCH)
            t4 = plsc.load_gather(tb, [idx + 4])
            p4 = plsc.load_gather(pb, [idx + 4])
            p9 = plsc.load_gather(pb, [idx + 9])
            objm = t4 > 0.0
            accn = accn + jnp.where(objm, 0.0, p4 * p4 + p9 * p9)
            rows = g * 16 + iota
            pos = cnt + lax.cumsum(objm.astype(jnp.int32), axis=0) - 1
            plsc.store_scatter(oidx, [jnp.maximum(pos, 0)], rows, mask=objm)
            return accn, cnt + jnp.sum(objm.astype(jnp.int32))

        acc_no, cnt = lax.fori_loop(0, GROUPS, phase_a,
                                    (acc_no, jnp.int32(0)))

        def phase_b(j, acch):
            lanes = j * 16 + iota
            valid = lanes < cnt
            lanec = jnp.minimum(lanes, cnt - 1)
            r = plsc.load_gather(oidx, [lanec])
            b = r * NCH

            def g(buf, ch):
                return plsc.load_gather(buf, [b + ch])

            t0, t1, t2, t3 = g(tb, 0), g(tb, 1), g(tb, 2), g(tb, 3)
            tx1, ty1, tx2, ty2 = _xyxy(t0, t1, t2, t3)
            area2 = (tx2 - tx1) * (ty2 - ty1)

            def iou_of(x, y, w, h):
                x1, y1, x2, y2 = _xyxy(x, y, w, h)
                iw = jnp.maximum(jnp.minimum(x2, tx2) - jnp.maximum(x1, tx1), 0.0)
                ih = jnp.maximum(jnp.minimum(y2, ty2) - jnp.maximum(y1, ty1), 0.0)
                inter = iw * ih
                area1 = (x2 - x1) * (y2 - y1)
                return inter / (area1 + area2 - inter)

            p0, p1, p2, p3, p4 = g(pb, 0), g(pb, 1), g(pb, 2), g(pb, 3), g(pb, 4)
            p5, p6, p7, p8, p9 = g(pb, 5), g(pb, 6), g(pb, 7), g(pb, 8), g(pb, 9)
            iou0 = iou_of(p0, p1, p2, p3)
            iou1 = iou_of(p5, p6, p7, p8)
            sel = iou1 > iou0
            maxiou = jnp.maximum(iou0, iou1)

            def pick(a, bb):
                return jnp.where(sel, bb, a)

            px, py = pick(p0, p5), pick(p1, p6)
            pw, ph = pick(p2, p7), pick(p3, p8)
            pc = pick(p4, p9)
            t5, t6, t7, t8 = g(tb, 5), g(tb, 6), g(tb, 7), g(tb, 8)
            qx, qy = pick(t0, t5), pick(t1, t6)
            qw, qh = pick(t2, t7), pick(t3, t8)

            dx, dy = px - qx, py - qy
            lxy = dx * dx + dy * dy
            dw = _sqrt16(pw) - _sqrt16(qw)
            dh = _sqrt16(ph) - _sqrt16(qh)
            lwh = dw * dw + dh * dh
            do = pc - maxiou
            lobj = do * do
            lcls = jnp.zeros((16,), jnp.float32)
            for k in range(10, 30):
                d = g(pb, k) - g(tb, k)
                lcls = lcls + d * d
            contrib = 5.0 * (lxy + lwh) + lobj + lcls
            return acch + jnp.where(valid, contrib, 0.0)

        acc_hv = lax.fori_loop(0, (cnt + 15) // 16, phase_b, acc_hv)

    obuf[...] = (acc_hv + 0.5 * acc_no) * (1.0 / 4096.0)
    pltpu.sync_copy(obuf, out_hbm.at[wid])


@jax.jit
def _sc_loss(p_flat, t_flat):
    mesh = plsc.VectorSubcoreMesh(core_axis_name="c", subcore_axis_name="s")
    run = functools.partial(
        pl.kernel,
        mesh=mesh,
        compiler_params=pltpu.CompilerParams(use_tc_tiling_on_sc=False,
                                             needs_layout_passes=False),
        out_type=jax.ShapeDtypeStruct((NW, 16), jnp.float32),
        scratch_types=[
            pltpu.VMEM((CHUNK_WORDS,), jnp.float32),
            pltpu.VMEM((CHUNK_WORDS,), jnp.float32),
            pltpu.VMEM((CHUNK_WORDS,), jnp.float32),
            pltpu.VMEM((CHUNK_WORDS,), jnp.float32),
            pltpu.VMEM((CHUNK_ROWS,), jnp.int32),
            pltpu.VMEM((16,), jnp.float32),
            pltpu.SemaphoreType.DMA,
            pltpu.SemaphoreType.DMA,
            pltpu.SemaphoreType.DMA,
            pltpu.SemaphoreType.DMA,
        ],
    )(_body)
    return run(p_flat, t_flat)


def kernel(pred_tensor, target_tensor):
    parts = _sc_loss(pred_tensor.reshape(-1), target_tensor.reshape(-1))
    return jnp.sum(parts)


# speedup vs baseline: 14.5016x; 4.4529x over previous
import functools

import jax
import jax.numpy as jnp
from jax import lax
from jax.experimental import pallas as pl
from jax.experimental.pallas import tpu as pltpu
from jax.experimental.pallas import tpu_sc as plsc

S = 7.0
NCH = 30
NB = 4096
NW = 32
BPW = NB // NW
NSLAB = 49
GROUPS = BPW // 16


def _sqrt16(x):
    i = lax.bitcast_convert_type(x, jnp.int32)
    y = lax.bitcast_convert_type(jnp.int32(0x5F3759DF) - (i >> 1), jnp.float32)
    for _ in range(3):
        y = y * (1.5 - 0.5 * x * y * y)
    return x * y


def _xyxy(x, y, w, h):
    cx = x / S
    cy = y / S
    return cx - 0.5 * w, cy - 0.5 * h, cx + 0.5 * w, cy + 0.5 * h


def _group_terms(pb, tb, k):
    sl = pl.ds(k * 16, 16)

    def p(c):
        return pb[c, sl]

    def t(c):
        return tb[c, sl]

    conf = t(4)
    objm = conf > 0.0
    p4, p9 = p(4), p(9)
    no_term = jnp.where(objm, 0.0, p4 * p4 + p9 * p9)

    t0, t1, t2, t3 = t(0), t(1), t(2), t(3)
    tx1, ty1, tx2, ty2 = _xyxy(t0, t1, t2, t3)
    area2 = (tx2 - tx1) * (ty2 - ty1)

    def iou_of(x, y, w, h):
        x1, y1, x2, y2 = _xyxy(x, y, w, h)
        iw = jnp.maximum(jnp.minimum(x2, tx2) - jnp.maximum(x1, tx1), 0.0)
        ih = jnp.maximum(jnp.minimum(y2, ty2) - jnp.maximum(y1, ty1), 0.0)
        inter = iw * ih
        area1 = (x2 - x1) * (y2 - y1)
        return inter / (area1 + area2 - inter)

    p0, p1, p2, p3 = p(0), p(1), p(2), p(3)
    p5, p6, p7, p8 = p(5), p(6), p(7), p(8)
    iou0 = iou_of(p0, p1, p2, p3)
    iou1 = iou_of(p5, p6, p7, p8)
    sel = iou1 > iou0
    maxiou = jnp.maximum(iou0, iou1)

    def pick(a, b):
        return jnp.where(sel, b, a)

    px, py = pick(p0, p5), pick(p1, p6)
    pw, ph = pick(p2, p7), pick(p3, p8)
    pc = pick(p4, p9)
    qx, qy = pick(t0, t(5)), pick(t1, t(6))
    qw, qh = pick(t2, t(7)), pick(t3, t(8))

    dx, dy = px - qx, py - qy
    lxy = dx * dx + dy * dy
    dw = _sqrt16(pw) - _sqrt16(qw)
    dh = _sqrt16(ph) - _sqrt16(qh)
    lwh = dw * dw + dh * dh
    do = pc - maxiou
    lobj = do * do
    lcls = jnp.zeros((16,), jnp.float32)
    for c in range(10, 30):
        d = p(c) - t(c)
        lcls = lcls + d * d
    heavy = jnp.where(objm, 5.0 * (lxy + lwh) + lobj + lcls, 0.0)
    return no_term, heavy


def _body(p_hbm, t_hbm, out_hbm, pb0, pb1, tb0, tb1, obuf,
          sp0, sp1, st0, st1):
    wid = lax.axis_index("s") * 2 + lax.axis_index("c")
    b0 = wid * BPW
    pbufs = (pb0, pb1)
    tbufs = (tb0, tb1)
    psems = (sp0, sp1)
    tsems = (st0, st1)

    def copies(g, slot):
        s1 = g // 7
        s2 = g % 7
        cp = pltpu.make_async_copy(
            p_hbm.at[s1, s2, :, pl.ds(b0, BPW)], pbufs[slot], psems[slot])
        ct = pltpu.make_async_copy(
            t_hbm.at[s1, s2, :, pl.ds(b0, BPW)], tbufs[slot], tsems[slot])
        return cp, ct

    def start(g, slot):
        cp, ct = copies(g, slot)
        cp.start()
        ct.start()

    def finish(g, slot):
        cp, ct = copies(g, slot)
        cp.wait()
        ct.wait()

    def do_slab(g, slot, acc_no, acc_hv):
        finish(g, slot)
        for k in range(GROUPS):
            no_term, heavy = _group_terms(pbufs[slot], tbufs[slot], k)
            acc_no = acc_no + no_term
            acc_hv = acc_hv + heavy
        return acc_no, acc_hv

    start(0, 0)
    start(1, 1)

    def pair(i, carry):
        acc_no, acc_hv = carry
        g = 2 * i
        acc_no, acc_hv = do_slab(g, 0, acc_no, acc_hv)
        start(g + 2, 0)
        acc_no, acc_hv = do_slab(g + 1, 1, acc_no, acc_hv)

        @pl.when(g + 3 < NSLAB)
        def _():
            start(g + 3, 1)

        return acc_no, acc_hv

    zero = jnp.zeros((16,), jnp.float32)
    acc_no, acc_hv = lax.fori_loop(0, (NSLAB - 1) // 2, pair, (zero, zero))
    acc_no, acc_hv = do_slab(NSLAB - 1, 0, acc_no, acc_hv)

    obuf[...] = (acc_hv + 0.5 * acc_no) * (1.0 / float(NB))
    pltpu.sync_copy(obuf, out_hbm.at[pl.ds(wid * 16, 16)])


@jax.jit
def _sc_loss(p4d, t4d):
    mesh = plsc.VectorSubcoreMesh(core_axis_name="c", subcore_axis_name="s")
    run = functools.partial(
        pl.kernel,
        mesh=mesh,
        compiler_params=pltpu.CompilerParams(use_tc_tiling_on_sc=True),
        out_type=jax.ShapeDtypeStruct((NW * 16,), jnp.float32),
        scratch_types=[
            pltpu.VMEM((NCH, BPW), jnp.float32),
            pltpu.VMEM((NCH, BPW), jnp.float32),
            pltpu.VMEM((NCH, BPW), jnp.float32),
            pltpu.VMEM((NCH, BPW), jnp.float32),
            pltpu.VMEM((16,), jnp.float32),
            pltpu.SemaphoreType.DMA,
            pltpu.SemaphoreType.DMA,
            pltpu.SemaphoreType.DMA,
            pltpu.SemaphoreType.DMA,
        ],
    )(_body)
    return run(p4d, t4d)


def kernel(pred_tensor, target_tensor):
    p4d = lax.transpose(pred_tensor, (1, 2, 3, 0))
    t4d = lax.transpose(target_tensor, (1, 2, 3, 0))
    parts = _sc_loss(p4d, t4d)
    return jnp.sum(parts)
